# baseline (speedup 1.0000x reference)
import functools

import jax
import jax.numpy as jnp
from jax import lax
from jax.experimental import pallas as pl
from jax.experimental.pallas import tpu as pltpu
from jax.experimental.pallas import tpu_sc as plsc

_N, _C = 32768, 128
_L = 16
_NGRP = _C // _L
_QUART = 4
_RPW = _N // _QUART
_CHUNK = 2048
_NCH = _RPW // _CHUNK
_BLK = 16
_NBLK = _CHUNK // _BLK
_CAP = 256
_BIS = 30


def _sc_body(inv_hbm, nea_hbm, x_hbm, o_hbm,
             buf, cand, blkm, vtmp, stage_min, stage_cnt, stage_cand):
    c = lax.axis_index("c")
    s = lax.axis_index("s")
    grp = c * (_NGRP // 2) + s // _QUART
    quart = s % _QUART
    col0 = grp * _L
    row0 = quart * _RPW

    pltpu.sync_copy(inv_hbm, vtmp)
    inv = vtmp[...][0]
    pltpu.sync_copy(nea_hbm, vtmp)
    nea = vtmp[...][0]
    lane = lax.iota(jnp.int32, _L)

    def initc(r, _):
        cand[r] = jnp.full((_L,), jnp.inf, jnp.float32)
        return 0
    lax.fori_loop(0, _CAP, initc, 0, unroll=4)

    def blockmin(b):
        m = buf[b * _BLK]
        for r in range(1, _BLK):
            m = jnp.minimum(m, buf[b * _BLK + r])
        return m

    def phase1(ch, carry):
        runmin, cnt = carry
        pltpu.sync_copy(
            x_hbm.at[pl.ds(row0 + ch * _CHUNK, _CHUNK), pl.ds(col0, _L)],
            buf)

        def minstep(b, rm):
            m = blockmin(b)
            blkm[b] = m
            return jnp.minimum(rm, m)
        runmin = lax.fori_loop(0, _NBLK, minstep, runmin)

        thr = runmin + inv

        def flagstep(b, cnt):
            bm = blkm[b]
            def do_scatter(cnt):
                cc = cnt
                for r in range(_BLK):
                    v = buf[b * _BLK + r]
                    m = jnp.logical_and(v < thr, cc < _CAP)
                    plsc.store_scatter(cand, [cc, lane], v, mask=m)
                    cc = cc + jnp.where(m, 1, 0)
                return cc
            return lax.cond(jnp.any(bm < thr), do_scatter, lambda k: k, cnt)
        cnt = lax.fori_loop(0, _NBLK, flagstep, cnt)
        return runmin, cnt

    runmin = jnp.full((_L,), jnp.inf, jnp.float32)
    cnt0 = jnp.zeros((_L,), jnp.int32)
    runmin, cnt = lax.fori_loop(0, _NCH, phase1, (runmin, cnt0))

    vtmp[...] = runmin
    pltpu.sync_copy(vtmp, stage_min.at[s])
    vtmp[...] = cnt.astype(jnp.float32)
    pltpu.sync_copy(vtmp, stage_cnt.at[s])
    pltpu.sync_copy(cand, stage_cand.at[s])
    plsc.subcore_barrier()

    base = (s // _QUART) * _QUART
    mn = jnp.full((_L,), jnp.inf, jnp.float32)
    cmax = jnp.zeros((_L,), jnp.float32)
    for j in range(_QUART):
        pltpu.sync_copy(stage_min.at[base + j], vtmp)
        mn = jnp.minimum(mn, vtmp[...])
        pltpu.sync_copy(stage_cnt.at[base + j], vtmp)
        cmax = jnp.maximum(cmax, vtmp[...])
        pltpu.sync_copy(stage_cand.at[base + j],
                        buf.at[pl.ds(j * _CAP, _CAP)])
    nrows = jnp.minimum(lax.reduce_max(cmax, (0,)),
                        jnp.float32(_CAP)).astype(jnp.int32)

    def gsum(sv):
        def seg(j, acc):
            def rstep(r, acc):
                return acc + jnp.minimum(buf[j * _CAP + r] + sv, 0.0)
            return lax.fori_loop(0, nrows, rstep, acc)
        return lax.fori_loop(0, _QUART, seg,
                             jnp.zeros((_L,), jnp.float32))

    s_lo = (-mn) - inv
    s_hi = -mn
    g_lo = jnp.full((_L,), -1e30, jnp.float32)
    g_hi = jnp.zeros((_L,), jnp.float32)

    def bis(_, st):
        lo, hi, glo, ghi = st
        mid = 0.5 * (lo + hi)
        gm = gsum(mid)
        big = gm <= -inv
        return (jnp.where(big, mid, lo), jnp.where(big, hi, mid),
                jnp.where(big, gm, glo), jnp.where(big, ghi, gm))
    s_lo, s_hi, g_lo, g_hi = lax.fori_loop(
        0, _BIS, bis, (s_lo, s_hi, g_lo, g_hi))
    denom = jnp.maximum(g_hi - g_lo, 1e-30)
    sv = s_lo + ((-inv) - g_lo) * (s_hi - s_lo) / denom

    def outch(ch, _):
        pltpu.sync_copy(
            x_hbm.at[pl.ds(row0 + ch * _CHUNK, _CHUNK), pl.ds(col0, _L)],
            buf)
        def rstep(r, _):
            buf[r] = nea * jnp.minimum(buf[r] + sv, 0.0)
            return 0
        lax.fori_loop(0, _CHUNK, rstep, 0, unroll=8)
        pltpu.sync_copy(
            buf,
            o_hbm.at[pl.ds(row0 + ch * _CHUNK, _CHUNK), pl.ds(col0, _L)])
        return 0
    lax.fori_loop(0, _NCH, outch, 0)


def kernel(x, a):
    inv = jnp.broadcast_to(jnp.exp(-a).astype(jnp.float32), (_L,))
    nea = jnp.broadcast_to((-jnp.exp(a)).astype(jnp.float32), (_L,))
    mesh = plsc.VectorSubcoreMesh(core_axis_name="c", subcore_axis_name="s")
    f = pl.kernel(
        _sc_body,
        out_type=jax.ShapeDtypeStruct((_N, _C), jnp.float32),
        mesh=mesh,
        compiler_params=pltpu.CompilerParams(use_tc_tiling_on_sc=False,
                                             needs_layout_passes=False),
        scratch_types=[
            pltpu.VMEM((_CHUNK, _L), jnp.float32),
            pltpu.VMEM((_CAP, _L), jnp.float32),
            pltpu.VMEM((_NBLK, _L), jnp.float32),
            pltpu.VMEM((_L,), jnp.float32),
            pltpu.VMEM_SHARED((16, _L), jnp.float32),
            pltpu.VMEM_SHARED((16, _L), jnp.float32),
            pltpu.VMEM_SHARED((16, _CAP, _L), jnp.float32),
        ],
    )
    return f(inv, nea, x)


# speedup vs baseline: 1.2349x; 1.2349x over previous
import jax
import jax.numpy as jnp
from jax import lax
from jax.experimental import pallas as pl
from jax.experimental.pallas import tpu as pltpu
from jax.experimental.pallas import tpu_sc as plsc

_N, _C = 32768, 128
_L = 16
_NGRP = _C // _L
_QUART = 4
_RPW = _N // _QUART
_CHUNK = 2048
_NCH = _RPW // _CHUNK
_BLK = 16
_NBLK = _CHUNK // _BLK
_CAP = 256
_BIS = 16


def _sc_body(inv_hbm, nea_hbm, x_hbm, o_hbm,
             bufa, bufb, cand, blkm, vtmp,
             sia, sib, soa, sob,
             stage_min, stage_cnt, stage_cand):
    c = lax.axis_index("c")
    s = lax.axis_index("s")
    grp = c * (_NGRP // 2) + s // _QUART
    quart = s % _QUART
    col0 = grp * _L
    row0 = quart * _RPW

    bufs = (bufa, bufb)
    isems = (sia, sib)
    osems = (soa, sob)

    def chunk_src(ch):
        return x_hbm.at[pl.ds(row0 + ch * _CHUNK, _CHUNK), pl.ds(col0, _L)]

    def chunk_dst(ch):
        return o_hbm.at[pl.ds(row0 + ch * _CHUNK, _CHUNK), pl.ds(col0, _L)]

    pltpu.sync_copy(inv_hbm, vtmp)
    inv = vtmp[...][0]
    pltpu.sync_copy(nea_hbm, vtmp)
    nea = vtmp[...][0]
    lane = lax.iota(jnp.int32, _L)

    def initc(r, _):
        cand[r] = jnp.full((_L,), jnp.inf, jnp.float32)
        return 0
    lax.fori_loop(0, _CAP, initc, 0, unroll=4)

    def blockmin(buf, b):
        m = buf[b * _BLK]
        for r in range(1, _BLK):
            m = jnp.minimum(m, buf[b * _BLK + r])
        return m

    runmin = jnp.full((_L,), jnp.inf, jnp.float32)
    cnt = jnp.zeros((_L,), jnp.int32)
    pltpu.make_async_copy(chunk_src(0), bufs[0], isems[0]).start()
    for ch in range(_NCH):
        buf = bufs[ch % 2]
        if ch + 1 < _NCH:
            pltpu.make_async_copy(
                chunk_src(ch + 1), bufs[(ch + 1) % 2],
                isems[(ch + 1) % 2]).start()
        pltpu.make_async_copy(chunk_src(ch), buf, isems[ch % 2]).wait()

        def minstep(b, rm):
            m = blockmin(buf, b)
            blkm[b] = m
            return jnp.minimum(rm, m)
        runmin = lax.fori_loop(0, _NBLK, minstep, runmin)

        thr = runmin + inv

        def flagstep(b, cnt):
            bm = blkm[b]
            def do_scatter(cc):
                for r in range(_BLK):
                    v = buf[b * _BLK + r]
                    m = jnp.logical_and(v < thr, cc < _CAP)
                    plsc.store_scatter(cand, [cc, lane], v, mask=m)
                    cc = cc + jnp.where(m, 1, 0)
                return cc
            return lax.cond(jnp.any(bm < thr), do_scatter, lambda k: k, cnt)
        cnt = lax.fori_loop(0, _NBLK, flagstep, cnt)

    vtmp[...] = runmin
    pltpu.sync_copy(vtmp, stage_min.at[s])
    vtmp[...] = cnt.astype(jnp.float32)
    pltpu.sync_copy(vtmp, stage_cnt.at[s])
    pltpu.sync_copy(cand, stage_cand.at[s])
    plsc.subcore_barrier()

    base = (s // _QUART) * _QUART
    mn = jnp.full((_L,), jnp.inf, jnp.float32)
    cmax = jnp.zeros((_L,), jnp.float32)
    for j in range(_QUART):
        pltpu.sync_copy(stage_min.at[base + j], vtmp)
        mn = jnp.minimum(mn, vtmp[...])
        pltpu.sync_copy(stage_cnt.at[base + j], vtmp)
        cmax = jnp.maximum(cmax, vtmp[...])
        pltpu.sync_copy(stage_cand.at[base + j],
                        bufa.at[pl.ds(j * _CAP, _CAP)])
    nrows = jnp.minimum(lax.reduce_max(cmax, (0,)),
                        jnp.float32(_CAP)).astype(jnp.int32)

    def gsum(sv):
        def seg(j, acc):
            def rstep(r, acc):
                return acc + jnp.minimum(bufa[j * _CAP + r] + sv, 0.0)
            return lax.fori_loop(0, nrows, rstep, acc)
        return lax.fori_loop(0, _QUART, seg,
                             jnp.zeros((_L,), jnp.float32))

    s_lo = (-mn) - inv
    s_hi = -mn
    g_lo = jnp.full((_L,), -1e30, jnp.float32)
    g_hi = jnp.zeros((_L,), jnp.float32)

    def bis(_, st):
        lo, hi, glo, ghi = st
        mid = 0.5 * (lo + hi)
        gm = gsum(mid)
        big = gm <= -inv
        return (jnp.where(big, mid, lo), jnp.where(big, hi, mid),
                jnp.where(big, gm, glo), jnp.where(big, ghi, gm))
    s_lo, s_hi, g_lo, g_hi = lax.fori_loop(
        0, _BIS, bis, (s_lo, s_hi, g_lo, g_hi))
    denom = jnp.maximum(g_hi - g_lo, 1e-30)
    sv = s_lo + ((-inv) - g_lo) * (s_hi - s_lo) / denom

    pltpu.make_async_copy(chunk_src(0), bufs[0], isems[0]).start()
    for ch in range(_NCH):
        buf = bufs[ch % 2]
        nxt = (ch + 1) % 2
        if ch + 1 < _NCH:
            if ch >= 1:
                pltpu.make_async_copy(
                    bufs[nxt], chunk_dst(ch - 1), osems[nxt]).wait()
            pltpu.make_async_copy(
                chunk_src(ch + 1), bufs[nxt], isems[nxt]).start()
        pltpu.make_async_copy(chunk_src(ch), buf, isems[ch % 2]).wait()

        def rstep(r, _):
            buf[r] = nea * jnp.minimum(buf[r] + sv, 0.0)
            return 0
        lax.fori_loop(0, _CHUNK, rstep, 0, unroll=8)

        pltpu.make_async_copy(buf, chunk_dst(ch), osems[ch % 2]).start()
    for ch in (_NCH - 2, _NCH - 1):
        pltpu.make_async_copy(
            bufs[ch % 2], chunk_dst(ch), osems[ch % 2]).wait()


def kernel(x, a):
    inv = jnp.broadcast_to(jnp.exp(-a).astype(jnp.float32), (_L,))
    nea = jnp.broadcast_to((-jnp.exp(a)).astype(jnp.float32), (_L,))
    mesh = plsc.VectorSubcoreMesh(core_axis_name="c", subcore_axis_name="s")
    f = pl.kernel(
        _sc_body,
        out_type=jax.ShapeDtypeStruct((_N, _C), jnp.float32),
        mesh=mesh,
        compiler_params=pltpu.CompilerParams(use_tc_tiling_on_sc=False,
                                             needs_layout_passes=False),
        scratch_types=[
            pltpu.VMEM((_CHUNK, _L), jnp.float32),
            pltpu.VMEM((_CHUNK, _L), jnp.float32),
            pltpu.VMEM((_CAP, _L), jnp.float32),
            pltpu.VMEM((_NBLK, _L), jnp.float32),
            pltpu.VMEM((_L,), jnp.float32),
            pltpu.SemaphoreType.DMA,
            pltpu.SemaphoreType.DMA,
            pltpu.SemaphoreType.DMA,
            pltpu.SemaphoreType.DMA,
            pltpu.VMEM_SHARED((16, _L), jnp.float32),
            pltpu.VMEM_SHARED((16, _L), jnp.float32),
            pltpu.VMEM_SHARED((16, _CAP, _L), jnp.float32),
        ],
    )
    return f(inv, nea, x)


# speedup vs baseline: 1.2844x; 1.0401x over previous
import jax
import jax.numpy as jnp
from jax import lax
from jax.experimental import pallas as pl
from jax.experimental.pallas import tpu as pltpu
from jax.experimental.pallas import tpu_sc as plsc

_N, _C = 32768, 128
_L = 16
_NGRP = _C // _L
_QUART = 4
_RPW = _N // _QUART
_CHUNK = 2048
_NCH = _RPW // _CHUNK
_BLK = 16
_NBLK = _CHUNK // _BLK
_CAP = 256
_BIS = 16


def _sc_body(inv_hbm, nea_hbm, x_hbm, o_hbm,
             bufa, bufb, cand, blkm, vtmp,
             sia, sib, soa, sob,
             stage_min, stage_cnt, stage_cand):
    c = lax.axis_index("c")
    s = lax.axis_index("s")
    grp = c * (_NGRP // 2) + s // _QUART
    quart = s % _QUART
    col0 = grp * _L
    row0 = quart * _RPW

    bufs = (bufa, bufb)
    isems = (sia, sib)
    osems = (soa, sob)

    def chunk_src(ch):
        return x_hbm.at[pl.ds(row0 + ch * _CHUNK, _CHUNK), pl.ds(col0, _L)]

    def chunk_dst(ch):
        return o_hbm.at[pl.ds(row0 + ch * _CHUNK, _CHUNK), pl.ds(col0, _L)]

    pltpu.sync_copy(inv_hbm, vtmp)
    inv = vtmp[...][0]
    pltpu.sync_copy(nea_hbm, vtmp)
    nea = vtmp[...][0]
    lane = lax.iota(jnp.int32, _L)

    def initc(r, _):
        cand[r] = jnp.full((_L,), jnp.inf, jnp.float32)
        return 0
    lax.fori_loop(0, _CAP, initc, 0, unroll=4)

    def blockmin(buf, b):
        m = buf[b * _BLK]
        for r in range(1, _BLK):
            m = jnp.minimum(m, buf[b * _BLK + r])
        return m

    runmin = jnp.full((_L,), jnp.inf, jnp.float32)
    cnt = jnp.zeros((_L,), jnp.int32)
    pltpu.make_async_copy(chunk_src(0), bufs[0], isems[0]).start()
    for ch in range(_NCH):
        buf = bufs[ch % 2]
        if ch + 1 < _NCH:
            pltpu.make_async_copy(
                chunk_src(ch + 1), bufs[(ch + 1) % 2],
                isems[(ch + 1) % 2]).start()
        pltpu.make_async_copy(chunk_src(ch), buf, isems[ch % 2]).wait()

        def minstep(b, rm):
            m = blockmin(buf, b)
            blkm[b] = m
            return jnp.minimum(rm, m)
        runmin = lax.fori_loop(0, _NBLK, minstep, runmin)

        thr = runmin + inv

        def flagstep(b, cnt):
            bm = blkm[b]
            def do_scatter(cc):
                for r in range(_BLK):
                    v = buf[b * _BLK + r]
                    m = jnp.logical_and(v < thr, cc < _CAP)
                    plsc.store_scatter(cand, [cc, lane], v, mask=m)
                    cc = cc + jnp.where(m, 1, 0)
                return cc
            return lax.cond(jnp.any(bm < thr), do_scatter, lambda k: k, cnt)
        cnt = lax.fori_loop(0, _NBLK, flagstep, cnt)

    vtmp[...] = runmin
    pltpu.sync_copy(vtmp, stage_min.at[s])
    vtmp[...] = cnt.astype(jnp.float32)
    pltpu.sync_copy(vtmp, stage_cnt.at[s])
    pltpu.sync_copy(cand, stage_cand.at[s])
    plsc.subcore_barrier()

    base = (s // _QUART) * _QUART
    mn = jnp.full((_L,), jnp.inf, jnp.float32)
    cmax = jnp.zeros((_L,), jnp.float32)
    for j in range(_QUART):
        pltpu.sync_copy(stage_min.at[base + j], vtmp)
        mn = jnp.minimum(mn, vtmp[...])
        pltpu.sync_copy(stage_cnt.at[base + j], vtmp)
        cmax = jnp.maximum(cmax, vtmp[...])
        pltpu.sync_copy(stage_cand.at[base + j],
                        bufa.at[pl.ds(j * _CAP, _CAP)])
    nrows = jnp.minimum(lax.reduce_max(cmax, (0,)),
                        jnp.float32(_CAP)).astype(jnp.int32)

    nrow8 = (nrows + 7) // 8

    def gsum(sv):
        def seg(j, acc):
            def rstep(r8, acc):
                for t in range(8):
                    acc = acc + jnp.minimum(
                        bufa[j * _CAP + r8 * 8 + t] + sv, 0.0)
                return acc
            return lax.fori_loop(0, nrow8, rstep, acc)
        return lax.fori_loop(0, _QUART, seg,
                             jnp.zeros((_L,), jnp.float32))

    s_lo = (-mn) - inv
    s_hi = -mn
    g_lo = jnp.full((_L,), -1e30, jnp.float32)
    g_hi = jnp.zeros((_L,), jnp.float32)

    def bis(_, st):
        lo, hi, glo, ghi = st
        mid = 0.5 * (lo + hi)
        gm = gsum(mid)
        big = gm <= -inv
        return (jnp.where(big, mid, lo), jnp.where(big, hi, mid),
                jnp.where(big, gm, glo), jnp.where(big, ghi, gm))
    s_lo, s_hi, g_lo, g_hi = lax.fori_loop(
        0, _BIS, bis, (s_lo, s_hi, g_lo, g_hi))
    denom = jnp.maximum(g_hi - g_lo, 1e-30)
    sv = s_lo + ((-inv) - g_lo) * (s_hi - s_lo) / denom

    pltpu.make_async_copy(chunk_src(0), bufs[0], isems[0]).start()
    for ch in range(_NCH):
        buf = bufs[ch % 2]
        nxt = (ch + 1) % 2
        if ch + 1 < _NCH:
            if ch >= 1:
                pltpu.make_async_copy(
                    bufs[nxt], chunk_dst(ch - 1), osems[nxt]).wait()
            pltpu.make_async_copy(
                chunk_src(ch + 1), bufs[nxt], isems[nxt]).start()
        pltpu.make_async_copy(chunk_src(ch), buf, isems[ch % 2]).wait()

        def rstep(r, _):
            buf[r] = nea * jnp.minimum(buf[r] + sv, 0.0)
            return 0
        lax.fori_loop(0, _CHUNK, rstep, 0, unroll=8)

        pltpu.make_async_copy(buf, chunk_dst(ch), osems[ch % 2]).start()
    for ch in (_NCH - 2, _NCH - 1):
        pltpu.make_async_copy(
            bufs[ch % 2], chunk_dst(ch), osems[ch % 2]).wait()


def kernel(x, a):
    inv = jnp.broadcast_to(jnp.exp(-a).astype(jnp.float32), (_L,))
    nea = jnp.broadcast_to((-jnp.exp(a)).astype(jnp.float32), (_L,))
    mesh = plsc.VectorSubcoreMesh(core_axis_name="c", subcore_axis_name="s")
    f = pl.kernel(
        _sc_body,
        out_type=jax.ShapeDtypeStruct((_N, _C), jnp.float32),
        mesh=mesh,
        compiler_params=pltpu.CompilerParams(use_tc_tiling_on_sc=False,
                                             needs_layout_passes=False),
        scratch_types=[
            pltpu.VMEM((_CHUNK, _L), jnp.float32),
            pltpu.VMEM((_CHUNK, _L), jnp.float32),
            pltpu.VMEM((_CAP, _L), jnp.float32),
            pltpu.VMEM((_NBLK, _L), jnp.float32),
            pltpu.VMEM((_L,), jnp.float32),
            pltpu.SemaphoreType.DMA,
            pltpu.SemaphoreType.DMA,
            pltpu.SemaphoreType.DMA,
            pltpu.SemaphoreType.DMA,
            pltpu.VMEM_SHARED((16, _L), jnp.float32),
            pltpu.VMEM_SHARED((16, _L), jnp.float32),
            pltpu.VMEM_SHARED((16, _CAP, _L), jnp.float32),
        ],
    )
    return f(inv, nea, x)
